# Initial kernel scaffold; baseline (speedup 1.0000x reference)
import jax
import jax.numpy as jnp
from jax.experimental import pallas as pl


def kernel(binary_mask):
    raise NotImplementedError("write your pallas kernel here")


# speedup vs baseline: 5.8140x; 5.8140x over previous
import functools

import jax
import jax.numpy as jnp
from jax.experimental import pallas as pl

_K = 5
_H = 512
_W = 512


def _kmeans_body(mask_ref, scores_ref, out_ref):
    m = mask_ref[0]
    yi = jax.lax.broadcasted_iota(jnp.int32, (_H, _W), 0)
    xi = jax.lax.broadcasted_iota(jnp.int32, (_H, _W), 1)
    y = yi.astype(jnp.float32)
    x = xi.astype(jnp.float32)
    pidx = yi * _W + xi
    big = jnp.int32(2 ** 30)

    fg = m != 0.0
    c2 = y * y + x * x
    count = jnp.sum(jnp.where(fg, 1.0, 0.0))

    s = scores_ref[0] + jnp.where(fg, 10.0, 0.0)
    init_cy = []
    init_cx = []
    for _ in range(_K):
        mx = jnp.max(s)
        p0 = jnp.min(jnp.where(s == mx, pidx, big))
        init_cy.append((p0 // _W).astype(jnp.float32))
        init_cx.append((p0 % _W).astype(jnp.float32))
        s = jnp.where(pidx == p0, -jnp.inf, s)

    def _bf(v):
        return v.astype(jnp.bfloat16).astype(jnp.float32)

    yb = _bf(y)
    xb = _bf(x)

    def distances(cys, cxs):
        ds = []
        for k in range(_K):
            cy, cx = cys[k], cxs[k]
            cent2 = cy * cy + cx * cx
            dot = yb * _bf(cy) + xb * _bf(cx)
            d2 = (c2 + cent2) - 2.0 * dot
            ds.append(jnp.sqrt(jnp.maximum(d2, 0.0)))
        return ds

    def argmin5(ds):
        best = ds[0]
        bk = jnp.zeros((_H, _W), jnp.int32)
        for k in range(1, _K):
            lt = ds[k] < best
            best = jnp.where(lt, ds[k], best)
            bk = jnp.where(lt, k, bk)
        return best, bk

    def update(it, carry):
        cys, cxs = carry
        ds = distances(cys, cxs)
        best, bk = argmin5(ds)
        mind = jnp.where(fg, best, -jnp.inf)
        mm = jnp.max(mind)
        pf = jnp.min(jnp.where(mind == mm, pidx, big))
        fy = (pf // _W).astype(jnp.float32)
        fx = (pf % _W).astype(jnp.float32)
        ncy = []
        ncx = []
        for k in range(_K):
            sel = fg & (bk == k)
            cnt = jnp.sum(jnp.where(sel, 1.0, 0.0))
            sy = jnp.sum(jnp.where(sel, yb, 0.0))
            sx = jnp.sum(jnp.where(sel, xb, 0.0))
            denom = jnp.maximum(cnt, 1.0)
            nonempty = cnt > 0.0
            ncy.append(jnp.where(nonempty, sy / denom, fy))
            ncx.append(jnp.where(nonempty, sx / denom, fx))
        return tuple(ncy), tuple(ncx)

    cys, cxs = jax.lax.fori_loop(
        0, 9, update, (tuple(init_cy), tuple(init_cx))
    )
    _, bk = argmin5(distances(cys, cxs))

    special = count <= float(_K)
    for k in range(_K):
        vk = jnp.where(fg & (bk == k), 1.0, 0.0)
        if k == 0:
            out_ref[0, k] = jnp.where(special, m, vk)
        else:
            out_ref[0, k] = jnp.where(special, 0.0, vk)


@jax.jit
def kernel(binary_mask):
    x = binary_mask
    if x.ndim == 5 and x.shape[1] == 1:
        x = x[:, 0]
    B, C, H, W = x.shape
    n = B * C
    masks = x.reshape(n, H, W)
    keys = jax.random.split(jax.random.key(42), n)
    scores = jax.vmap(lambda k: jax.random.uniform(k, (H * W,)))(keys)
    scores = scores.reshape(n, H, W)

    out = pl.pallas_call(
        _kmeans_body,
        grid=(n,),
        in_specs=[
            pl.BlockSpec((1, H, W), lambda i: (i, 0, 0)),
            pl.BlockSpec((1, H, W), lambda i: (i, 0, 0)),
        ],
        out_specs=pl.BlockSpec((1, _K, H, W), lambda i: (i, 0, 0, 0)),
        out_shape=jax.ShapeDtypeStruct((n, _K, H, W), masks.dtype),
    )(masks, scores)
    return out.reshape(B, C, _K, H, W)


# speedup vs baseline: 7.7444x; 1.3320x over previous
import functools

import jax
import jax.numpy as jnp
from jax.experimental import pallas as pl
from jax.experimental.pallas import tpu as pltpu

_K = 5
_H = 512
_W = 512


def _kmeans_body(mask_ref, scores_ref, out_ref):
    m = mask_ref[0]
    yi = jax.lax.broadcasted_iota(jnp.int32, (_H, _W), 0)
    xi = jax.lax.broadcasted_iota(jnp.int32, (_H, _W), 1)
    y = yi.astype(jnp.float32)
    x = xi.astype(jnp.float32)
    pidx = yi * _W + xi
    big = jnp.int32(2 ** 30)

    fg = m != 0.0
    c2 = y * y + x * x
    count = jnp.sum(jnp.where(fg, 1.0, 0.0))

    s = scores_ref[0] + jnp.where(fg, 10.0, 0.0)
    init_cy = []
    init_cx = []
    for _ in range(_K):
        mx = jnp.max(s)
        p0 = jnp.min(jnp.where(s == mx, pidx, big))
        init_cy.append((p0 // _W).astype(jnp.float32))
        init_cx.append((p0 % _W).astype(jnp.float32))
        s = jnp.where(pidx == p0, -jnp.inf, s)

    def _bf(v):
        return v.astype(jnp.bfloat16).astype(jnp.float32)

    yb = _bf(y)
    xb = _bf(x)

    fgf = jnp.where(fg, 1.0, 0.0)

    def distances(cys, cxs, with_sqrt):
        ds = []
        for k in range(_K):
            cy, cx = cys[k], cxs[k]
            cent2 = cy * cy + cx * cx
            dot = yb * _bf(cy) + xb * _bf(cx)
            d2 = (c2 + cent2) - 2.0 * dot
            d2 = jnp.maximum(d2, 0.0)
            ds.append(jnp.sqrt(d2) if with_sqrt else d2)
        return ds

    def argmin5(ds):
        best = ds[0]
        bk = jnp.zeros((_H, _W), jnp.int32)
        for k in range(1, _K):
            lt = ds[k] < best
            best = jnp.where(lt, ds[k], best)
            bk = jnp.where(lt, k, bk)
        return best, bk

    def update(it, carry):
        cys, cxs = carry
        ds = distances(cys, cxs, with_sqrt=False)
        best, bk = argmin5(ds)
        mind = jnp.where(fg, best, -jnp.inf)
        mm = jnp.max(mind)
        pf = jnp.min(jnp.where(mind == mm, pidx, big))
        fy = (pf // _W).astype(jnp.float32)
        fx = (pf % _W).astype(jnp.float32)
        ncy = []
        ncx = []
        for k in range(_K):
            self_k = jnp.where(bk == k, fgf, 0.0)
            cnt = jnp.sum(self_k)
            sy = jnp.sum(self_k * yb)
            sx = jnp.sum(self_k * xb)
            denom = jnp.maximum(cnt, 1.0)
            nonempty = cnt > 0.0
            ncy.append(jnp.where(nonempty, sy / denom, fy))
            ncx.append(jnp.where(nonempty, sx / denom, fx))
        return tuple(ncy), tuple(ncx)

    cys, cxs = jax.lax.fori_loop(
        0, 9, update, (tuple(init_cy), tuple(init_cx))
    )
    _, bk = argmin5(distances(cys, cxs, with_sqrt=True))

    special = count <= float(_K)
    for k in range(_K):
        vk = jnp.where(fg & (bk == k), 1.0, 0.0)
        if k == 0:
            out_ref[0, k] = jnp.where(special, m, vk)
        else:
            out_ref[0, k] = jnp.where(special, 0.0, vk)


@jax.jit
def kernel(binary_mask):
    x = binary_mask
    if x.ndim == 5 and x.shape[1] == 1:
        x = x[:, 0]
    B, C, H, W = x.shape
    n = B * C
    masks = x.reshape(n, H, W)
    keys = jax.random.split(jax.random.key(42), n)
    scores = jax.vmap(lambda k: jax.random.uniform(k, (H * W,)))(keys)
    scores = scores.reshape(n, H, W)

    out = pl.pallas_call(
        _kmeans_body,
        grid=(n,),
        in_specs=[
            pl.BlockSpec((1, H, W), lambda i: (i, 0, 0)),
            pl.BlockSpec((1, H, W), lambda i: (i, 0, 0)),
        ],
        out_specs=pl.BlockSpec((1, _K, H, W), lambda i: (i, 0, 0, 0)),
        out_shape=jax.ShapeDtypeStruct((n, _K, H, W), masks.dtype),
        compiler_params=pltpu.CompilerParams(
            dimension_semantics=("parallel",),
        ),
    )(masks, scores)
    return out.reshape(B, C, _K, H, W)


# speedup vs baseline: 10.9434x; 1.4131x over previous
import jax
import jax.numpy as jnp
from jax.experimental import pallas as pl
from jax.experimental.pallas import tpu as pltpu

_K = 5
_H = 512
_W = 512
_NM = 2


def _bf(v):
    return v.astype(jnp.bfloat16).astype(jnp.float32)


def _kmeans_body(mask_ref, scores_ref, out_ref):
    yi = jax.lax.broadcasted_iota(jnp.int32, (_H, _W), 0)
    xi = jax.lax.broadcasted_iota(jnp.int32, (_H, _W), 1)
    y = yi.astype(jnp.float32)
    x = xi.astype(jnp.float32)
    pidx = yi * _W + xi
    big = jnp.int32(2 ** 30)
    c2 = y * y + x * x
    yb = _bf(y)
    xb = _bf(x)

    ms = [mask_ref[j] for j in range(_NM)]
    fgs = [m != 0.0 for m in ms]
    fgfs = [jnp.where(fg, 1.0, 0.0) for fg in fgs]
    counts = [jnp.sum(fgf) for fgf in fgfs]

    init_cy = [[] for _ in range(_NM)]
    init_cx = [[] for _ in range(_NM)]
    ss = [scores_ref[j] + jnp.where(fgs[j], 10.0, 0.0) for j in range(_NM)]
    for _ in range(_K):
        for j in range(_NM):
            mx = jnp.max(ss[j])
            p0 = jnp.min(jnp.where(ss[j] == mx, pidx, big))
            init_cy[j].append((p0 // _W).astype(jnp.float32))
            init_cx[j].append((p0 % _W).astype(jnp.float32))
            ss[j] = jnp.where(pidx == p0, -jnp.inf, ss[j])

    def distances(cys, cxs, with_sqrt):
        ds = []
        for k in range(_K):
            cy, cx = cys[k], cxs[k]
            cent2 = cy * cy + cx * cx
            dot = yb * _bf(cy) + xb * _bf(cx)
            d2 = (c2 + cent2) - 2.0 * dot
            d2 = jnp.maximum(d2, 0.0)
            ds.append(jnp.sqrt(d2) if with_sqrt else d2)
        return ds

    def argmin5(ds):
        best = ds[0]
        bk = jnp.zeros((_H, _W), jnp.int32)
        for k in range(1, _K):
            lt = ds[k] < best
            best = jnp.where(lt, ds[k], best)
            bk = jnp.where(lt, k, bk)
        return best, bk

    ones_row = jnp.ones((8, _H), jnp.float32)
    ones_col = jnp.ones((_W, 8), jnp.float32)
    xv = jax.lax.broadcasted_iota(jnp.int32, (8, _W), 1).astype(jnp.float32)
    yv = jax.lax.broadcasted_iota(jnp.int32, (_H, 8), 0).astype(jnp.float32)
    xvb = _bf(xv[0:1, :])
    yvb = _bf(yv[:, 0:1])

    def update_one(fg, fgf, cys, cxs):
        ds = distances(cys, cxs, with_sqrt=False)
        best, bk = argmin5(ds)
        mind = jnp.where(fg, best, -jnp.inf)
        mm = jnp.max(mind)
        pf = jnp.min(jnp.where(mind == mm, pidx, big))
        fy = (pf // _W).astype(jnp.float32)
        fx = (pf % _W).astype(jnp.float32)
        ncy = []
        ncx = []
        for k in range(_K):
            self_k = jnp.where(bk == k, fgf, 0.0)
            colcnt = jnp.dot(ones_row, self_k)
            rowcnt = jnp.dot(self_k, ones_col)
            cnt = jnp.sum(colcnt[0:1, :])
            sx = jnp.sum(colcnt[0:1, :] * xvb)
            sy = jnp.sum(rowcnt[:, 0:1] * yvb)
            denom = jnp.maximum(cnt, 1.0)
            nonempty = cnt > 0.0
            ncy.append(jnp.where(nonempty, sy / denom, fy))
            ncx.append(jnp.where(nonempty, sx / denom, fx))
        return tuple(ncy), tuple(ncx)

    def update(it, carry):
        return tuple(
            update_one(fgs[j], fgfs[j], carry[j][0], carry[j][1])
            for j in range(_NM)
        )

    carry0 = tuple((tuple(init_cy[j]), tuple(init_cx[j])) for j in range(_NM))
    carry = jax.lax.fori_loop(0, 9, update, carry0)

    for j in range(_NM):
        cys, cxs = carry[j]
        _, bk = argmin5(distances(cys, cxs, with_sqrt=True))
        special = counts[j] <= float(_K)
        for k in range(_K):
            vk = jnp.where(fgs[j] & (bk == k), 1.0, 0.0)
            if k == 0:
                out_ref[j, k] = jnp.where(special, ms[j], vk)
            else:
                out_ref[j, k] = jnp.where(special, 0.0, vk)


@jax.jit
def kernel(binary_mask):
    x = binary_mask
    if x.ndim == 5 and x.shape[1] == 1:
        x = x[:, 0]
    B, C, H, W = x.shape
    n = B * C
    masks = x.reshape(n, H, W)
    keys = jax.random.split(jax.random.key(42), n)
    scores = jax.vmap(lambda k: jax.random.uniform(k, (H * W,)))(keys)
    scores = scores.reshape(n, H, W)

    steps = n // _NM
    out = pl.pallas_call(
        _kmeans_body,
        grid=(steps,),
        in_specs=[
            pl.BlockSpec((_NM, H, W), lambda i: (i, 0, 0)),
            pl.BlockSpec((_NM, H, W), lambda i: (i, 0, 0)),
        ],
        out_specs=pl.BlockSpec((_NM, _K, H, W), lambda i: (i, 0, 0, 0)),
        out_shape=jax.ShapeDtypeStruct((n, _K, H, W), masks.dtype),
        compiler_params=pltpu.CompilerParams(
            dimension_semantics=("parallel",),
        ),
    )(masks, scores)
    return out.reshape(B, C, _K, H, W)
